# baseline (speedup 1.0000x reference)
import jax
import jax.numpy as jnp
from jax.experimental import pallas as pl
from jax.experimental.pallas import tpu as pltpu

_BN_EPS = 1e-5
_C = 3


def _ssum(a):
    return jnp.sum(jnp.sum(a, axis=1, keepdims=True), axis=0, keepdims=True)


def _rsqrt1(x_scalar):
    return jax.lax.rsqrt(jnp.full((1, 1), x_scalar, jnp.float32))[0, 0]


def _fused_kernel(w1_ref, b1_ref, w2_ref, b2_ref, x_ref, o_ref,
                  xbuf_ref, acc_ref, pp_ref, *, n_tiles, pixels):
    p = pl.program_id(0)
    t = pl.program_id(1)
    bb = x_ref.shape[0]
    hw = x_ref.shape[1] // _C
    row = t * bb

    @pl.when(p == 0)
    def _phase0():
        x0 = jnp.maximum(x_ref[:, 0 * hw:1 * hw], 0.0)
        x1 = jnp.maximum(x_ref[:, 1 * hw:2 * hw], 0.0)
        x2 = jnp.maximum(x_ref[:, 2 * hw:3 * hw], 0.0)

        xbuf_ref[pl.ds(row, bb), 0 * hw:1 * hw] = x0
        xbuf_ref[pl.ds(row, bb), 1 * hw:2 * hw] = x1
        xbuf_ref[pl.ds(row, bb), 2 * hw:3 * hw] = x2

        @pl.when(t == 0)
        def _():
            acc_ref[...] = jnp.zeros_like(acc_ref)

        acc_ref[:, 0:9] += jnp.concatenate(
            [
                _ssum(x0), _ssum(x1), _ssum(x2),
                _ssum(x0 * x0), _ssum(x1 * x1), _ssum(x2 * x2),
                _ssum(x0 * x1), _ssum(x1 * x2), _ssum(x0 * x2),
            ],
            axis=1,
        )

    @pl.when(p == 1)
    def _phase1():
        @pl.when(t == 0)
        def _fold():
            a = acc_ref[...]
            s = [a[0, 0], a[0, 1], a[0, 2]]
            m = [[a[0, 3], a[0, 6], a[0, 8]],
                 [a[0, 6], a[0, 4], a[0, 7]],
                 [a[0, 8], a[0, 7], a[0, 5]]]
            w1 = [[w1_ref[j, i] for i in range(_C)] for j in range(_C)]
            w2 = [[w2_ref[j, i] for i in range(_C)] for j in range(_C)]
            b1 = [b1_ref[j] for j in range(_C)]
            b2 = [b2_ref[j] for j in range(_C)]
            fp = float(pixels)

            mean, inv = [], []
            for j in range(_C):
                w1s = sum(w1[j][i] * s[i] for i in range(_C))
                mean_j = (w1s + fp * b1[j]) / fp
                q = sum(w1[j][i] * w1[j][k] * m[i][k]
                        for i in range(_C) for k in range(_C))
                sum_v2 = q + 2.0 * b1[j] * w1s + fp * b1[j] * b1[j]
                var_j = jnp.maximum(sum_v2 / fp - mean_j * mean_j, 0.0)
                mean.append(mean_j)
                inv.append(_rsqrt1(var_j + _BN_EPS))

            w22 = [[sum(w2[j][i] * w2[i][k] for i in range(_C))
                    for k in range(_C)] for j in range(_C)]
            b22 = [sum(w2[j][i] * b2[i] for i in range(_C)) + b2[j]
                   for j in range(_C)]
            g = [inv[i] * b1[i] + 1.0 - mean[i] * inv[i] for i in range(_C)]
            for j in range(_C):
                for k in range(_C):
                    pp_ref[3 * j + k] = sum(
                        w22[j][i] * inv[i] * w1[i][k] for i in range(_C))
                pp_ref[9 + j] = sum(
                    w22[j][i] * g[i] for i in range(_C)) + b22[j]

        y0 = xbuf_ref[pl.ds(row, bb), 0 * hw:1 * hw]
        y1 = xbuf_ref[pl.ds(row, bb), 1 * hw:2 * hw]
        y2 = xbuf_ref[pl.ds(row, bb), 2 * hw:3 * hw]

        q = pp_ref
        t0 = jnp.maximum(q[0] * y0 + q[1] * y1 + q[2] * y2 + q[9], 0.0)
        t1 = jnp.maximum(q[3] * y0 + q[4] * y1 + q[5] * y2 + q[10], 0.0)
        t2 = jnp.maximum(q[6] * y0 + q[7] * y1 + q[8] * y2 + q[11], 0.0)

        o_ref[:, 0 * hw:1 * hw] = (w2_ref[0, 0] * t0 + w2_ref[0, 1] * t1 +
                                   w2_ref[0, 2] * t2 + b2_ref[0])
        o_ref[:, 1 * hw:2 * hw] = (w2_ref[1, 0] * t0 + w2_ref[1, 1] * t1 +
                                   w2_ref[1, 2] * t2 + b2_ref[1])
        o_ref[:, 2 * hw:3 * hw] = (w2_ref[2, 0] * t0 + w2_ref[2, 1] * t1 +
                                   w2_ref[2, 2] * t2 + b2_ref[2])


def kernel(x_nchw, w1, b1, w2, b2):
    import functools

    N, c_in, H, W = x_nchw.shape
    assert c_in == _C
    HW = H * W
    P = N * HW

    x2d = x_nchw.reshape(N, _C * HW)

    B = 1
    for cand in (32, 16, 8, 4, 2):
        if N % cand == 0:
            B = cand
            break
    n_tiles = N // B

    smem_spec = pl.BlockSpec(memory_space=pltpu.MemorySpace.SMEM)
    x_spec = pl.BlockSpec(
        (B, _C * HW),
        lambda p, t: (jax.lax.select(p == 0, t, n_tiles - 1), 0))
    o_spec = pl.BlockSpec(
        (B, _C * HW),
        lambda p, t: (jax.lax.select(p == 0, 0, t), 0))

    out2d = pl.pallas_call(
        functools.partial(_fused_kernel, n_tiles=n_tiles, pixels=P),
        out_shape=jax.ShapeDtypeStruct((N, _C * HW), jnp.float32),
        grid=(2, n_tiles),
        in_specs=[smem_spec, smem_spec, smem_spec, smem_spec, x_spec],
        out_specs=o_spec,
        scratch_shapes=[
            pltpu.VMEM((N, _C * HW), jnp.float32),
            pltpu.VMEM((1, 128), jnp.float32),
            pltpu.SMEM((12,), jnp.float32),
        ],
        compiler_params=pltpu.CompilerParams(
            dimension_semantics=("arbitrary", "arbitrary"),
            vmem_limit_bytes=56 * 1024 * 1024),
        cost_estimate=pl.CostEstimate(
            flops=55 * P, transcendentals=3, bytes_accessed=8 * _C * P),
    )(w1.astype(jnp.float32), b1.astype(jnp.float32),
      w2.astype(jnp.float32), b2.astype(jnp.float32), x2d)

    return out2d.reshape(N, _C, H, W)


# speedup vs baseline: 1.1010x; 1.1010x over previous
import jax
import jax.numpy as jnp
from jax.experimental import pallas as pl
from jax.experimental.pallas import tpu as pltpu

_BN_EPS = 1e-5
_C = 3


def _ssum(a):
    return jnp.sum(jnp.sum(a, axis=1, keepdims=True), axis=0, keepdims=True)


def _rsqrt1(x_scalar):
    return jax.lax.rsqrt(jnp.full((1, 1), x_scalar, jnp.float32))[0, 0]


def _fused_kernel(w1_ref, b1_ref, w2_ref, b2_ref, x_ref, o_ref,
                  xbuf_ref, acc_ref, pp_ref, *, n_tiles, pixels):
    p = pl.program_id(0)
    t = pl.program_id(1)
    bb = x_ref.shape[0]
    hw = x_ref.shape[1] // _C
    row = t * bb

    @pl.when(p == 0)
    def _phase0():
        x0 = jnp.maximum(x_ref[:, 0 * hw:1 * hw], 0.0)
        x1 = jnp.maximum(x_ref[:, 1 * hw:2 * hw], 0.0)
        x2 = jnp.maximum(x_ref[:, 2 * hw:3 * hw], 0.0)

        xbuf_ref[pl.ds(row, bb), 0 * hw:1 * hw] = x0
        xbuf_ref[pl.ds(row, bb), 1 * hw:2 * hw] = x1
        xbuf_ref[pl.ds(row, bb), 2 * hw:3 * hw] = x2

        @pl.when(t == 0)
        def _():
            acc_ref[...] = jnp.zeros_like(acc_ref)

        acc_ref[:, 0:9] += jnp.concatenate(
            [
                _ssum(x0), _ssum(x1), _ssum(x2),
                _ssum(x0 * x0), _ssum(x1 * x1), _ssum(x2 * x2),
                _ssum(x0 * x1), _ssum(x1 * x2), _ssum(x0 * x2),
            ],
            axis=1,
        )

    @pl.when(p == 1)
    def _phase1():
        @pl.when(t == 0)
        def _fold():
            a = acc_ref[...]
            s = [a[0, 0], a[0, 1], a[0, 2]]
            m = [[a[0, 3], a[0, 6], a[0, 8]],
                 [a[0, 6], a[0, 4], a[0, 7]],
                 [a[0, 8], a[0, 7], a[0, 5]]]
            w1 = [[w1_ref[j, i] for i in range(_C)] for j in range(_C)]
            w2 = [[w2_ref[j, i] for i in range(_C)] for j in range(_C)]
            b1 = [b1_ref[j] for j in range(_C)]
            b2 = [b2_ref[j] for j in range(_C)]
            fp = float(pixels)

            mean, inv = [], []
            for j in range(_C):
                w1s = sum(w1[j][i] * s[i] for i in range(_C))
                mean_j = (w1s + fp * b1[j]) / fp
                q = sum(w1[j][i] * w1[j][k] * m[i][k]
                        for i in range(_C) for k in range(_C))
                sum_v2 = q + 2.0 * b1[j] * w1s + fp * b1[j] * b1[j]
                var_j = jnp.maximum(sum_v2 / fp - mean_j * mean_j, 0.0)
                mean.append(mean_j)
                inv.append(_rsqrt1(var_j + _BN_EPS))

            w22 = [[sum(w2[j][i] * w2[i][k] for i in range(_C))
                    for k in range(_C)] for j in range(_C)]
            b22 = [sum(w2[j][i] * b2[i] for i in range(_C)) + b2[j]
                   for j in range(_C)]
            g = [inv[i] * b1[i] + 1.0 - mean[i] * inv[i] for i in range(_C)]
            for j in range(_C):
                for k in range(_C):
                    pp_ref[3 * j + k] = sum(
                        w22[j][i] * inv[i] * w1[i][k] for i in range(_C))
                pp_ref[9 + j] = sum(
                    w22[j][i] * g[i] for i in range(_C)) + b22[j]

        y0 = xbuf_ref[pl.ds(row, bb), 0 * hw:1 * hw]
        y1 = xbuf_ref[pl.ds(row, bb), 1 * hw:2 * hw]
        y2 = xbuf_ref[pl.ds(row, bb), 2 * hw:3 * hw]

        q = pp_ref
        t0 = jnp.maximum(q[0] * y0 + q[1] * y1 + q[2] * y2 + q[9], 0.0)
        t1 = jnp.maximum(q[3] * y0 + q[4] * y1 + q[5] * y2 + q[10], 0.0)
        t2 = jnp.maximum(q[6] * y0 + q[7] * y1 + q[8] * y2 + q[11], 0.0)

        o_ref[:, 0 * hw:1 * hw] = (w2_ref[0, 0] * t0 + w2_ref[0, 1] * t1 +
                                   w2_ref[0, 2] * t2 + b2_ref[0])
        o_ref[:, 1 * hw:2 * hw] = (w2_ref[1, 0] * t0 + w2_ref[1, 1] * t1 +
                                   w2_ref[1, 2] * t2 + b2_ref[1])
        o_ref[:, 2 * hw:3 * hw] = (w2_ref[2, 0] * t0 + w2_ref[2, 1] * t1 +
                                   w2_ref[2, 2] * t2 + b2_ref[2])


def kernel(x_nchw, w1, b1, w2, b2):
    import functools

    N, c_in, H, W = x_nchw.shape
    assert c_in == _C
    HW = H * W
    P = N * HW

    x2d = x_nchw.reshape(N, _C * HW)

    B = 1
    for cand in (128, 64, 32, 16, 8, 4, 2):
        if N % cand == 0:
            B = cand
            break
    n_tiles = N // B

    smem_spec = pl.BlockSpec(memory_space=pltpu.MemorySpace.SMEM)
    x_spec = pl.BlockSpec(
        (B, _C * HW),
        lambda p, t: (jax.lax.select(p == 0, t, n_tiles - 1), 0))
    o_spec = pl.BlockSpec(
        (B, _C * HW),
        lambda p, t: (jax.lax.select(p == 0, 0, t), 0))

    out2d = pl.pallas_call(
        functools.partial(_fused_kernel, n_tiles=n_tiles, pixels=P),
        out_shape=jax.ShapeDtypeStruct((N, _C * HW), jnp.float32),
        grid=(2, n_tiles),
        in_specs=[smem_spec, smem_spec, smem_spec, smem_spec, x_spec],
        out_specs=o_spec,
        scratch_shapes=[
            pltpu.VMEM((N, _C * HW), jnp.float32),
            pltpu.VMEM((1, 128), jnp.float32),
            pltpu.SMEM((12,), jnp.float32),
        ],
        compiler_params=pltpu.CompilerParams(
            dimension_semantics=("arbitrary", "arbitrary"),
            vmem_limit_bytes=56 * 1024 * 1024),
        cost_estimate=pl.CostEstimate(
            flops=55 * P, transcendentals=3, bytes_accessed=8 * _C * P),
    )(w1.astype(jnp.float32), b1.astype(jnp.float32),
      w2.astype(jnp.float32), b2.astype(jnp.float32), x2d)

    return out2d.reshape(N, _C, H, W)
